# baseline (speedup 1.0000x reference)
import functools

import jax
import jax.numpy as jnp
from jax import lax
from jax.experimental import pallas as pl
from jax.experimental.pallas import tpu as pltpu
from jax.experimental.pallas import tpu_sc as plsc

N = 10000
E = 320000
D = 128

NC = 2
NS = 16
L = 16

CH = 80
EPT = E // (NC * NS)
NCHUNK = EPT // CH
NPAD = 10240
ROWS_PT = NPAD // NS


def _mlp_body(x_ref, w_ref, b_ref, o_ref):
    h = lax.dot_general(
        x_ref[...], w_ref[...], (((1,), (1,)), ((), ())),
        preferred_element_type=jnp.float32,
    )
    o_ref[...] = jnp.maximum(h + b_ref[...], 0.0)


def _mlp(x, W, b2):
    return pl.pallas_call(
        _mlp_body,
        grid=(10,),
        in_specs=[
            pl.BlockSpec((N // 10, D), lambda i: (i, 0)),
            pl.BlockSpec((D, D), lambda i: (0, 0)),
            pl.BlockSpec((1, D), lambda i: (0, 0)),
        ],
        out_specs=pl.BlockSpec((N // 10, D), lambda i: (i, 0)),
        out_shape=jax.ShapeDtypeStruct((N, D), jnp.float32),
    )(x, W, b2)


def _add_body(p_ref, o_ref):
    o_ref[...] = p_ref[0] + p_ref[1]


def _combine(partials):
    return pl.pallas_call(
        _add_body,
        grid=(10,),
        in_specs=[pl.BlockSpec((NC, N // 10, D), lambda i: (0, i, 0))],
        out_specs=pl.BlockSpec((N // 10, D), lambda i: (i, 0)),
        out_shape=jax.ShapeDtypeStruct((N, D), jnp.float32),
    )(partials)


NBUF = 4


@functools.partial(
    pl.kernel,
    out_type=jax.ShapeDtypeStruct((NC, NPAD, D), jnp.float32),
    mesh=plsc.VectorSubcoreMesh(core_axis_name="c", subcore_axis_name="s"),
    scratch_types=(
        [pltpu.VMEM((CH, D), jnp.float32)] * NBUF
        + [pltpu.VMEM((CH,), jnp.int32)] * NBUF
        + [pltpu.VMEM((CH,), jnp.int32)] * NBUF
        + [pltpu.VMEM((CH,), jnp.float32)] * NBUF
        + [pltpu.VMEM_SHARED((NPAD, D), jnp.float32)]
        + [pltpu.SemaphoreType.DMA] * (5 * NBUF)
    ),
)
def _edge_agg(src_hbm, dst_hbm, w_hbm, h_hbm, out_hbm, *refs):
    bufs = refs[0:NBUF]
    sch = refs[NBUF:2 * NBUF]
    dch = refs[2 * NBUF:3 * NBUF]
    wch = refs[3 * NBUF:4 * NBUF]
    acc = refs[4 * NBUF]
    sems = refs[4 * NBUF + 1:]
    gsem = sems[0:NBUF]
    ssem = sems[NBUF:2 * NBUF]
    srcsem = sems[2 * NBUF:3 * NBUF]
    dsem = sems[3 * NBUF:4 * NBUF]
    wsem = sems[4 * NBUF:5 * NBUF]

    c = lax.axis_index("c")
    s = lax.axis_index("s")
    wid = c * NS + s
    ebase = wid * EPT

    def _srccopy(g, k):
        return pltpu.make_async_copy(
            src_hbm.at[pl.ds(ebase + g * CH, CH)], sch[k], srcsem[k])

    def _dstcopy(g, k):
        return pltpu.make_async_copy(
            dst_hbm.at[pl.ds(ebase + g * CH, CH)], dch[k], dsem[k])

    def _wcopy(g, k):
        return pltpu.make_async_copy(
            w_hbm.at[pl.ds(ebase + g * CH, CH)], wch[k], wsem[k])

    def _gather(k):
        return pltpu.make_async_copy(h_hbm.at[sch[k]], bufs[k], gsem[k])

    def _scatter(k):
        return pltpu.make_async_copy(bufs[k], acc.at[dch[k]], ssem[k])


    def _compute(k):
        def _grp(gi, carry2):
            wvec = wch[k][pl.ds(gi * L, L)]
            for e16 in range(L):
                wspl = lax.gather(
                    wvec, jnp.full((L, 1), e16, jnp.int32),
                    lax.GatherDimensionNumbers(
                        offset_dims=(), collapsed_slice_dims=(0,),
                        start_index_map=(0,)),
                    slice_sizes=(1,),
                    mode=lax.GatherScatterMode.PROMISE_IN_BOUNDS)
                e = gi * L + e16
                for q in range(D // L):
                    bufs[k][e, pl.ds(q * L, L)] = (
                        bufs[k][e, pl.ds(q * L, L)] * wspl)
            return carry2

        lax.fori_loop(0, CH // L, _grp, 0)

    for k in range(NBUF):
        _srccopy(k, k).start()
    for k in range(3):
        _dstcopy(k, k).start()
        _wcopy(k, k).start()
    for k in range(3):
        _srccopy(k, k).wait()
        _gather(k).start()

    z16 = jnp.zeros((L,), jnp.float32)
    zb = bufs[NBUF - 1]

    def _zero(i, carry):
        r = i // (D // L)
        q = i % (D // L)
        zb[r, pl.ds(q * L, L)] = z16
        return carry

    lax.fori_loop(0, CH * (D // L), _zero, 0)
    for k in range(ROWS_PT // CH):
        pltpu.sync_copy(zb, acc.at[pl.ds(s * ROWS_PT + k * CH, CH)])
    plsc.subcore_barrier()

    def _quad(p, carry):
        for t in range(NBUF):
            g = NBUF * p + t
            k = t
            kn = (t + 3) % NBUF
            _gather(k).wait()

            @pl.when(g + 4 < NCHUNK)
            def _start_src():
                _srccopy(g + 4, k).start()

            @pl.when(g >= 1)
            def _wait_scatter():
                _scatter(kn).wait()

            @pl.when(g + 3 < NCHUNK)
            def _prefetch():
                _dstcopy(g + 3, kn).start()
                _wcopy(g + 3, kn).start()
                _srccopy(g + 3, kn).wait()
                _gather(kn).start()

            _wcopy(g, k).wait()
            _compute(k)
            _dstcopy(g, k).wait()
            _scatter(k).start(add=True)
        return carry

    lax.fori_loop(0, NCHUNK // NBUF, _quad, 0)

    glast = NCHUNK - 1
    kl = glast % NBUF
    _gather(kl).wait()
    _scatter((kl + 3) % NBUF).wait()
    _wcopy(glast, kl).wait()
    _compute(kl)
    _dstcopy(glast, kl).wait()
    _scatter(kl).start(add=True)
    _scatter(kl).wait()
    plsc.subcore_barrier()

    pltpu.sync_copy(acc.at[pl.ds(s * ROWS_PT, ROWS_PT)],
                    out_hbm.at[c, pl.ds(s * ROWS_PT, ROWS_PT)])


def kernel(x, edge_index, edge_w, W, b):
    h = _mlp(x, W, b.reshape(1, D))
    src = edge_index[0]
    dst = edge_index[1]
    partials = _edge_agg(src, dst, edge_w, h)
    return _combine(partials)


# speedup vs baseline: 1.0631x; 1.0631x over previous
import functools

import jax
import jax.numpy as jnp
from jax import lax
from jax.experimental import pallas as pl
from jax.experimental.pallas import tpu as pltpu
from jax.experimental.pallas import tpu_sc as plsc

N = 10000
E = 320000
D = 128

NC = 2
NS = 16
L = 16

CH = 80
EPT = E // (NC * NS)
NCHUNK = EPT // CH
NPAD = 10240
ROWS_PT = NPAD // NS


def _mlp_body(x_ref, w_ref, b_ref, o_ref):
    h = lax.dot_general(
        x_ref[...], w_ref[...], (((1,), (1,)), ((), ())),
        preferred_element_type=jnp.float32,
    )
    o_ref[...] = jnp.maximum(h + b_ref[...], 0.0)


def _mlp(x, W, b2):
    return pl.pallas_call(
        _mlp_body,
        grid=(10,),
        in_specs=[
            pl.BlockSpec((N // 10, D), lambda i: (i, 0)),
            pl.BlockSpec((D, D), lambda i: (0, 0)),
            pl.BlockSpec((1, D), lambda i: (0, 0)),
        ],
        out_specs=pl.BlockSpec((N // 10, D), lambda i: (i, 0)),
        out_shape=jax.ShapeDtypeStruct((N, D), jnp.float32),
    )(x, W, b2)


def _add_body(p_ref, o_ref):
    o_ref[...] = p_ref[0] + p_ref[1]


def _combine(partials):
    return pl.pallas_call(
        _add_body,
        grid=(10,),
        in_specs=[pl.BlockSpec((NC, N // 10, D), lambda i: (0, i, 0))],
        out_specs=pl.BlockSpec((N // 10, D), lambda i: (i, 0)),
        out_shape=jax.ShapeDtypeStruct((N, D), jnp.float32),
    )(partials)


NBUF = 4


@functools.partial(
    pl.kernel,
    out_type=jax.ShapeDtypeStruct((NC, NPAD, D), jnp.float32),
    mesh=plsc.VectorSubcoreMesh(core_axis_name="c", subcore_axis_name="s"),
    scratch_types=(
        [pltpu.VMEM((CH, D), jnp.float32)] * NBUF
        + [pltpu.VMEM((CH,), jnp.int32)] * NBUF
        + [pltpu.VMEM((CH,), jnp.int32)] * NBUF
        + [pltpu.VMEM((CH,), jnp.float32)] * NBUF
        + [pltpu.VMEM_SHARED((NPAD, D), jnp.float32)]
        + [pltpu.SemaphoreType.DMA] * (5 * NBUF)
    ),
)
def _edge_agg(edge_hbm, w_hbm, h_hbm, out_hbm, *refs):
    bufs = refs[0:NBUF]
    sch = refs[NBUF:2 * NBUF]
    dch = refs[2 * NBUF:3 * NBUF]
    wch = refs[3 * NBUF:4 * NBUF]
    acc = refs[4 * NBUF]
    sems = refs[4 * NBUF + 1:]
    gsem = sems[0:NBUF]
    ssem = sems[NBUF:2 * NBUF]
    srcsem = sems[2 * NBUF:3 * NBUF]
    dsem = sems[3 * NBUF:4 * NBUF]
    wsem = sems[4 * NBUF:5 * NBUF]

    c = lax.axis_index("c")
    s = lax.axis_index("s")
    wid = c * NS + s
    ebase = wid * EPT

    def _srccopy(g, k):
        return pltpu.make_async_copy(
            edge_hbm.at[pl.ds(ebase + g * CH, CH)], sch[k], srcsem[k])

    def _dstcopy(g, k):
        return pltpu.make_async_copy(
            edge_hbm.at[pl.ds(E + ebase + g * CH, CH)], dch[k], dsem[k])

    def _wcopy(g, k):
        return pltpu.make_async_copy(
            w_hbm.at[pl.ds(ebase + g * CH, CH)], wch[k], wsem[k])

    def _gather(k):
        return pltpu.make_async_copy(h_hbm.at[sch[k]], bufs[k], gsem[k])

    def _scatter(k):
        return pltpu.make_async_copy(bufs[k], acc.at[dch[k]], ssem[k])


    def _compute(k):
        def _grp(gi, carry2):
            wvec = wch[k][pl.ds(gi * L, L)]
            for e16 in range(L):
                wspl = lax.gather(
                    wvec, jnp.full((L, 1), e16, jnp.int32),
                    lax.GatherDimensionNumbers(
                        offset_dims=(), collapsed_slice_dims=(0,),
                        start_index_map=(0,)),
                    slice_sizes=(1,),
                    mode=lax.GatherScatterMode.PROMISE_IN_BOUNDS)
                e = gi * L + e16
                for q in range(D // L):
                    bufs[k][e, pl.ds(q * L, L)] = (
                        bufs[k][e, pl.ds(q * L, L)] * wspl)
            return carry2

        lax.fori_loop(0, CH // L, _grp, 0)

    for k in range(NBUF):
        _srccopy(k, k).start()
    for k in range(3):
        _dstcopy(k, k).start()
        _wcopy(k, k).start()
    for k in range(3):
        _srccopy(k, k).wait()
        _gather(k).start()

    z16 = jnp.zeros((L,), jnp.float32)
    zb = bufs[NBUF - 1]

    def _zero(i, carry):
        r = i // (D // L)
        q = i % (D // L)
        zb[r, pl.ds(q * L, L)] = z16
        return carry

    lax.fori_loop(0, CH * (D // L), _zero, 0)
    for k in range(ROWS_PT // CH):
        pltpu.sync_copy(zb, acc.at[pl.ds(s * ROWS_PT + k * CH, CH)])
    plsc.subcore_barrier()

    def _quad(p, carry):
        for t in range(NBUF):
            g = NBUF * p + t
            k = t
            kn = (t + 3) % NBUF
            _gather(k).wait()

            @pl.when(g + 4 < NCHUNK)
            def _start_src():
                _srccopy(g + 4, k).start()

            @pl.when(g >= 1)
            def _wait_scatter():
                _scatter(kn).wait()

            @pl.when(g + 3 < NCHUNK)
            def _prefetch():
                _dstcopy(g + 3, kn).start()
                _wcopy(g + 3, kn).start()
                _srccopy(g + 3, kn).wait()
                _gather(kn).start()

            _wcopy(g, k).wait()
            _compute(k)
            _dstcopy(g, k).wait()
            _scatter(k).start(add=True)
        return carry

    lax.fori_loop(0, NCHUNK // NBUF, _quad, 0)

    glast = NCHUNK - 1
    kl = glast % NBUF
    _gather(kl).wait()
    _scatter((kl + 3) % NBUF).wait()
    _wcopy(glast, kl).wait()
    _compute(kl)
    _dstcopy(glast, kl).wait()
    _scatter(kl).start(add=True)
    _scatter(kl).wait()
    plsc.subcore_barrier()

    pltpu.sync_copy(acc.at[pl.ds(s * ROWS_PT, ROWS_PT)],
                    out_hbm.at[c, pl.ds(s * ROWS_PT, ROWS_PT)])


def kernel(x, edge_index, edge_w, W, b):
    h = _mlp(x, W, b.reshape(1, D))
    partials = _edge_agg(edge_index.reshape(2 * E), edge_w, h)
    return _combine(partials)
